# baseline (speedup 1.0000x reference)
import functools

import jax
import jax.numpy as jnp
from jax import lax
from jax.experimental import pallas as pl
from jax.experimental.pallas import tpu as pltpu
from jax.experimental.pallas import tpu_sc as plsc

_B, _N = 64, 32768
_K = int(0.1 * _N)
_POS = _N - _K

_NC, _NS, _L = 2, 16, 16
_NW = _NC * _NS
_RPW = _B // _NW
_H12 = 4096
_H8 = 256


def _scan_hist(hist_ref, nbins, rank):
    def body(c, carry):
        cum, ans, before = carry
        h = hist_ref[pl.ds(c * _L, _L)]
        csg = plsc.cumsum(h) + cum
        lt = csg < rank
        ans = ans + jnp.sum(lt.astype(jnp.int32))
        before = before + jnp.sum(jnp.where(lt, h, 0))
        cum = cum + jnp.sum(h)
        return cum, ans, before
    _, ans, before = lax.fori_loop(
        0, nbins // _L, body,
        (jnp.int32(0), jnp.int32(0), jnp.int32(0)))
    return ans, before


def _zero_hist(hist_ref, nbins):
    def body(c, _):
        hist_ref[pl.ds(c * _L, _L)] = jnp.zeros((_L,), jnp.int32)
        return 0
    lax.fori_loop(0, nbins // _L, body, 0)


def _sc_threshold_body(x_hbm, thr_hbm, rowbuf, ubuf, hist, tbuf):
    INT_MIN = jnp.int32(-2147483648)
    wid = lax.axis_index("s") * _NC + lax.axis_index("c")
    iota = lax.iota(jnp.int32, _L)
    ones = jnp.ones((_L,), jnp.int32)

    tvec = jnp.zeros((_L,), jnp.float32)
    for rr in range(_RPW):
        row = wid * _RPW + rr
        pltpu.sync_copy(x_hbm.at[row], rowbuf)

        _zero_hist(hist, _H12)

        def p1_body(j, _):
            xv = rowbuf[pl.ds(j * _L, _L)]
            iv = plsc.bitcast(xv, jnp.int32)
            uv = iv ^ (lax.shift_right_arithmetic(iv, 31) | INT_MIN)
            ubuf[pl.ds(j * _L, _L)] = uv
            b = lax.shift_right_logical(uv, 20)
            plsc.addupdate_scatter(hist, [b], ones)
            return 0
        lax.fori_loop(0, _N // _L, p1_body, 0)

        ans1, before1 = _scan_hist(hist, _H12, _POS)
        rank2 = _POS - before1

        _zero_hist(hist, _H12)

        def p2_body(j, _):
            uv = ubuf[pl.ds(j * _L, _L)]
            match = lax.shift_right_logical(uv, 20) == ans1
            b = lax.shift_right_logical(uv, 8) & jnp.int32(0xFFF)
            plsc.addupdate_scatter(hist, [b], ones, mask=match)
            return 0
        lax.fori_loop(0, _N // _L, p2_body, 0)

        ans2, before2 = _scan_hist(hist, _H12, rank2)
        rank3 = rank2 - before2

        _zero_hist(hist, _H8)
        top24 = (ans1 << 12) | ans2

        def p3_body(j, _):
            uv = ubuf[pl.ds(j * _L, _L)]
            match = lax.shift_right_logical(uv, 8) == top24
            b = uv & jnp.int32(0xFF)
            plsc.addupdate_scatter(hist, [b], ones, mask=match)
            return 0
        lax.fori_loop(0, _N // _L, p3_body, 0)

        ans3, _ = _scan_hist(hist, _H8, rank3)

        u_thr = (ans1 << 20) | (ans2 << 8) | ans3
        i_thr = jnp.where(u_thr < 0, u_thr ^ INT_MIN, ~u_thr)
        fv = plsc.bitcast(lax.broadcast(i_thr, (_L,)), jnp.float32)
        tvec = jnp.where(iota == rr, fv, tvec)

    tbuf[...] = tvec
    pltpu.sync_copy(tbuf, thr_hbm.at[pl.ds(wid * _L, _L)])


def _make_sc_thresholds():
    mesh = plsc.VectorSubcoreMesh(core_axis_name="c", subcore_axis_name="s")
    return functools.partial(
        pl.kernel,
        out_type=jax.ShapeDtypeStruct((_NW * _L,), jnp.float32),
        mesh=mesh,
        compiler_params=pltpu.CompilerParams(needs_layout_passes=False),
        scratch_types=[
            pltpu.VMEM((_N,), jnp.float32),
            pltpu.VMEM((_N,), jnp.int32),
            pltpu.VMEM((_H12,), jnp.int32),
            pltpu.VMEM((_L,), jnp.float32),
        ],
    )(_sc_threshold_body)


_sc_thresholds = _make_sc_thresholds()


def _mask_blk(x_ref, t_ref, o_ref):
    x = x_ref[...]
    o_ref[...] = jnp.where(x > t_ref[...], x, jnp.float32(0.0))


_MASK_ROWS = 8


def _mask_call(x, thr):
    grid = _B // _MASK_ROWS
    return pl.pallas_call(
        _mask_blk,
        grid=(grid,),
        in_specs=[
            pl.BlockSpec((_MASK_ROWS, _N), lambda g: (g, 0)),
            pl.BlockSpec((_MASK_ROWS, 1), lambda g: (g, 0)),
        ],
        out_specs=pl.BlockSpec((_MASK_ROWS, _N), lambda g: (g, 0)),
        out_shape=jax.ShapeDtypeStruct((_B, _N), jnp.float32),
    )(x, thr)


@jax.jit
def kernel(x):
    thr512 = _sc_thresholds(x)
    thr = thr512.reshape(_NW, _L)[:, :_RPW].reshape(_B, 1)
    return _mask_call(x, thr)


# speedup vs baseline: 1.1547x; 1.1547x over previous
import functools

import jax
import jax.numpy as jnp
from jax import lax
from jax.experimental import pallas as pl
from jax.experimental.pallas import tpu as pltpu
from jax.experimental.pallas import tpu_sc as plsc

_B, _N = 64, 32768
_K = int(0.1 * _N)
_POS = _N - _K

_NC, _NS, _L = 2, 16, 16
_NW = _NC * _NS
_RPW = _B // _NW
_H12 = 4096
_H8 = 256


def _scan_hist(hist_ref, nbins, rank):
    def body(c, carry):
        cum, ans, before = carry
        h = hist_ref[pl.ds(c * _L, _L)]
        hist_ref[pl.ds(c * _L, _L)] = jnp.zeros((_L,), jnp.int32)
        csg = plsc.cumsum(h) + cum
        lt = csg < rank
        ans = ans + jnp.sum(lt.astype(jnp.int32))
        before = before + jnp.sum(jnp.where(lt, h, 0))
        cum = cum + jnp.sum(h)
        return cum, ans, before
    _, ans, before = lax.fori_loop(
        0, nbins // _L, body,
        (jnp.int32(0), jnp.int32(0), jnp.int32(0)), unroll=4)
    return ans, before


def _sc_body(x_hbm, out_hbm, rowbuf, ubuf, hist):
    INT_MIN = jnp.int32(-2147483648)
    wid = lax.axis_index("s") * _NC + lax.axis_index("c")
    ones = jnp.ones((_L,), jnp.int32)

    def z_body(c, _):
        hist[pl.ds(c * _L, _L)] = jnp.zeros((_L,), jnp.int32)
        return 0
    lax.fori_loop(0, _H12 // _L, z_body, 0, unroll=8)

    for rr in range(_RPW):
        row = wid * _RPW + rr
        pltpu.sync_copy(x_hbm.at[row], rowbuf)

        def p1_body(j, _):
            xv = rowbuf[pl.ds(j * _L, _L)]
            iv = plsc.bitcast(xv, jnp.int32)
            uv = iv ^ (lax.shift_right_arithmetic(iv, 31) | INT_MIN)
            ubuf[pl.ds(j * _L, _L)] = uv
            b = lax.shift_right_logical(uv, 20)
            plsc.addupdate_scatter(hist, [b], ones)
            return 0
        lax.fori_loop(0, _N // _L, p1_body, 0, unroll=8)

        ans1, before1 = _scan_hist(hist, _H12, _POS)
        rank2 = _POS - before1

        def p2_body(j, _):
            uv = ubuf[pl.ds(j * _L, _L)]
            match = lax.shift_right_logical(uv, 20) == ans1
            b = lax.shift_right_logical(uv, 8) & jnp.int32(0xFFF)
            plsc.addupdate_scatter(hist, [b], ones, mask=match)
            return 0
        lax.fori_loop(0, _N // _L, p2_body, 0, unroll=8)

        ans2, before2 = _scan_hist(hist, _H12, rank2)
        rank3 = rank2 - before2

        top24 = (ans1 << 12) | ans2

        def p3_body(j, _):
            uv = ubuf[pl.ds(j * _L, _L)]
            match = lax.shift_right_logical(uv, 8) == top24
            b = uv & jnp.int32(0xFF)
            plsc.addupdate_scatter(hist, [b], ones, mask=match)
            return 0
        lax.fori_loop(0, _N // _L, p3_body, 0, unroll=8)

        ans3, _ = _scan_hist(hist, _H8, rank3)

        u_thr = (ans1 << 20) | (ans2 << 8) | ans3
        i_thr = jnp.where(u_thr < 0, u_thr ^ INT_MIN, ~u_thr)
        thr = plsc.bitcast(lax.broadcast(i_thr, (_L,)), jnp.float32)

        def mk_body(j, _):
            xv = rowbuf[pl.ds(j * _L, _L)]
            rowbuf[pl.ds(j * _L, _L)] = jnp.where(
                xv > thr, xv, jnp.float32(0.0))
            return 0
        lax.fori_loop(0, _N // _L, mk_body, 0, unroll=8)

        pltpu.sync_copy(rowbuf, out_hbm.at[row])


def _make_sc_kernel():
    mesh = plsc.VectorSubcoreMesh(core_axis_name="c", subcore_axis_name="s")
    return functools.partial(
        pl.kernel,
        out_type=jax.ShapeDtypeStruct((_B, _N), jnp.float32),
        mesh=mesh,
        compiler_params=pltpu.CompilerParams(needs_layout_passes=False),
        scratch_types=[
            pltpu.VMEM((_N,), jnp.float32),
            pltpu.VMEM((_N,), jnp.int32),
            pltpu.VMEM((_H12,), jnp.int32),
        ],
    )(_sc_body)


_sc_kwinners = _make_sc_kernel()


@jax.jit
def kernel(x):
    return _sc_kwinners(x)
